# Initial kernel scaffold; baseline (speedup 1.0000x reference)
import jax
import jax.numpy as jnp
from jax.experimental import pallas as pl


def kernel(x, token_table, pos_table):
    raise NotImplementedError("write your pallas kernel here")


# speedup vs baseline: 1.3948x; 1.3948x over previous
import functools

import jax
import jax.numpy as jnp
from jax import lax
from jax.experimental import pallas as pl
from jax.experimental.pallas import tpu as pltpu
from jax.experimental.pallas import tpu_sc as plsc

B = 4096
T = 200
D = 32
N = B * T
NC = 2
NS = 16
NW = NC * NS
ROWS_PER_W = N // NW
CHUNK = 800
NCHUNK = ROWS_PER_W // CHUNK
REPS = CHUNK // T
HALF = D // 2


def _embed(x_hbm, tok_hbm, pos_hbm, out_hbm, idx_v, rows_v, pos_v, sem):
    wid = lax.axis_index("s") * NC + lax.axis_index("c")
    base = wid * ROWS_PER_W

    pltpu.sync_copy(pos_hbm, pos_v)

    def chunk_body(c, _):
        off = base + c * CHUNK
        pltpu.sync_copy(x_hbm.at[pl.ds(off, CHUNK)], idx_v)
        pltpu.async_copy(tok_hbm.at[idx_v], rows_v, sem).wait()

        def add_row(i, _):
            p0 = pos_v[i, pl.ds(0, HALF)]
            p1 = pos_v[i, pl.ds(HALF, HALF)]
            for rep in range(REPS):
                r = rep * T + i
                plsc.addupdate(rows_v.at[r, pl.ds(0, HALF)], p0)
                plsc.addupdate(rows_v.at[r, pl.ds(HALF, HALF)], p1)
            return 0

        lax.fori_loop(0, T, add_row, 0)
        pltpu.sync_copy(rows_v, out_hbm.at[pl.ds(off, CHUNK)])
        return 0

    lax.fori_loop(0, NCHUNK, chunk_body, 0)


def kernel(x, token_table, pos_table):
    xf = x.reshape(N).astype(jnp.int32)
    mesh = plsc.VectorSubcoreMesh(core_axis_name="c", subcore_axis_name="s")
    run = pl.kernel(
        _embed,
        out_type=jax.ShapeDtypeStruct((N, D), jnp.float32),
        mesh=mesh,
        scratch_types=[
            pltpu.VMEM((CHUNK,), jnp.int32),
            pltpu.VMEM((CHUNK, D), jnp.float32),
            pltpu.VMEM((T, D), jnp.float32),
            pltpu.SemaphoreType.DMA,
        ],
        compiler_params=pltpu.CompilerParams(use_tc_tiling_on_sc=False),
    )
    out = run(xf, token_table, pos_table)
    return out.reshape(B, T, D)


# speedup vs baseline: 1.4944x; 1.0713x over previous
import jax
import jax.numpy as jnp
from jax import lax
from jax.experimental import pallas as pl
from jax.experimental.pallas import tpu as pltpu
from jax.experimental.pallas import tpu_sc as plsc

B = 4096
T = 200
D = 32
N = B * T
NC = 2
NS = 16
NW = NC * NS
ROWS_PER_W = N // NW
CHUNK = 800
NCHUNK = ROWS_PER_W // CHUNK
REPS = CHUNK // T
HALF = D // 2
NBUF = 4
AHEAD = 2


def _embed(x_hbm, tok_hbm, pos_hbm, out_hbm, idx_v, rows_v, pos_v, *sems):
    gsem = sems[:NBUF]
    wsem = sems[NBUF:]
    wid = lax.axis_index("s") * NC + lax.axis_index("c")
    base = wid * ROWS_PER_W

    pltpu.sync_copy(pos_hbm, pos_v)

    def issue(c, b):
        off = base + c * CHUNK
        pltpu.sync_copy(x_hbm.at[pl.ds(off, CHUNK)], idx_v.at[b])
        pltpu.async_copy(tok_hbm.at[idx_v.at[b]], rows_v.at[b], gsem[b])

    def wait_gather(b):
        pltpu.make_async_copy(
            tok_hbm.at[idx_v.at[b]], rows_v.at[b], gsem[b]).wait()

    def start_writeback(c, b):
        off = base + c * CHUNK
        pltpu.async_copy(rows_v.at[b], out_hbm.at[pl.ds(off, CHUNK)], wsem[b])

    def wait_writeback(c, b):
        off = base + c * CHUNK
        pltpu.make_async_copy(
            rows_v.at[b], out_hbm.at[pl.ds(off, CHUNK)], wsem[b]).wait()

    def add_pos(b):
        def add_row(i, _):
            p0 = pos_v[i, pl.ds(0, HALF)]
            p1 = pos_v[i, pl.ds(HALF, HALF)]
            for rep in range(REPS):
                r = rep * T + i
                plsc.addupdate(rows_v.at[b, r, pl.ds(0, HALF)], p0)
                plsc.addupdate(rows_v.at[b, r, pl.ds(HALF, HALF)], p1)
            return 0

        lax.fori_loop(0, T, add_row, 0)

    for j in range(AHEAD):
        issue(j, j)

    def group_body(c0, _):
        for b in range(NBUF):
            c = c0 + b
            bi = (b + AHEAD) % NBUF
            ca = c + AHEAD

            @pl.when(ca < NCHUNK)
            def _():
                @pl.when(ca >= NBUF)
                def _():
                    wait_writeback(ca - NBUF, bi)

                issue(ca, bi)

            wait_gather(b)
            add_pos(b)
            start_writeback(c, b)
        return 0

    lax.fori_loop(0, NCHUNK // NBUF, lambda g, _: group_body(g * NBUF, 0), 0)

    for b in range(NBUF):
        wait_writeback(NCHUNK - NBUF + b, b)


def kernel(x, token_table, pos_table):
    xf = x.reshape(N).astype(jnp.int32)
    mesh = plsc.VectorSubcoreMesh(core_axis_name="c", subcore_axis_name="s")
    run = pl.kernel(
        _embed,
        out_type=jax.ShapeDtypeStruct((N, D), jnp.float32),
        mesh=mesh,
        scratch_types=[
            pltpu.VMEM((NBUF, CHUNK), jnp.int32),
            pltpu.VMEM((NBUF, CHUNK, D), jnp.float32),
            pltpu.VMEM((T, D), jnp.float32),
        ] + [pltpu.SemaphoreType.DMA] * (2 * NBUF),
        compiler_params=pltpu.CompilerParams(use_tc_tiling_on_sc=False),
    )
    out = run(xf, token_table, pos_table)
    return out.reshape(B, T, D)
